# baseline (speedup 1.0000x reference)
import functools

import jax
import jax.numpy as jnp
from jax.experimental import pallas as pl
from jax.experimental.pallas import tpu as pltpu


def _dg(a, b, dims):
    return (a, b, (dims, ((), ())))


def _D00(a, b, d):
    return jax.lax.dot_general(*_dg(a, b, d), preferred_element_type=jnp.float32)


def _D01(a, b, d):
    return jax.lax.dot_general(*_dg(a, b, d), preferred_element_type=jnp.float32)


def _D02(a, b, d):
    return jax.lax.dot_general(*_dg(a, b, d), preferred_element_type=jnp.float32)


def _D03(a, b, d):
    return jax.lax.dot_general(*_dg(a, b, d), preferred_element_type=jnp.float32)


def _D04(a, b, d):
    return jax.lax.dot_general(*_dg(a, b, d), preferred_element_type=jnp.float32)


def _D05(a, b, d):
    return jax.lax.dot_general(*_dg(a, b, d), preferred_element_type=jnp.float32)


def _D06(a, b, d):
    return jax.lax.dot_general(*_dg(a, b, d), preferred_element_type=jnp.float32)


def _D07(a, b, d):
    return jax.lax.dot_general(*_dg(a, b, d), preferred_element_type=jnp.float32)


def _D08(a, b, d):
    return jax.lax.dot_general(*_dg(a, b, d), preferred_element_type=jnp.float32)


def _D09(a, b, d):
    return jax.lax.dot_general(*_dg(a, b, d), preferred_element_type=jnp.float32)


def _fused_kernel(xt_ref, a_ref, feat_ref,
                  w1t_ref, b1_ref, w2_ref, b2_ref, fcwt_ref, fcb_ref,
                  wgi_ref, wgf_ref, wgo_ref, wgg_ref,
                  bi_ref, bf_ref, bo_ref, bg_ref,
                  wii_ref, wif_ref, wio_ref, wig_ref,
                  whi_ref, whf_ref, who_ref, whg_ref,
                  out_ref, c_ref, hN_ref,
                  feat_buf, feat_sem,
                  *, n_nodes, gh, seq_len, batch, hidden, in_dim):
    f32 = jnp.float32
    N, GH, T, B, H = n_nodes, gh, seq_len, batch, hidden
    NG = N * GH

    def feat_start(slot, t):
        pltpu.make_async_copy(feat_ref.at[t], feat_buf.at[slot],
                              feat_sem.at[slot]).start()

    def feat_wait(slot):
        pltpu.make_async_copy(feat_buf.at[slot], feat_buf.at[slot],
                              feat_sem.at[slot]).wait()

    feat_start(0, 0)
    feat_start(1, 1)

    col = jax.lax.broadcasted_iota(jnp.int32, (N, NG), 1)
    row = jax.lax.broadcasted_iota(jnp.int32, (N, NG), 0)
    mask = (col // GH == row).astype(f32)
    kcol = jax.lax.broadcasted_iota(jnp.int32, (GH, NG), 1)
    krow = jax.lax.broadcasted_iota(jnp.int32, (GH, NG), 0)
    ktile = (kcol % GH == krow).astype(f32)
    ones_row = jnp.ones((1, N), f32)

    wg = jnp.concatenate([wgi_ref[...], wgf_ref[...],
                          wgo_ref[...], wgg_ref[...]], axis=1)
    b4 = jnp.concatenate([bi_ref[...], bf_ref[...],
                          bo_ref[...], bg_ref[...]], axis=1)
    wx = jnp.concatenate([wii_ref[...], wif_ref[...],
                          wio_ref[...], wig_ref[...]], axis=1)
    wh = jnp.concatenate([whi_ref[...], whf_ref[...],
                          who_ref[...], whg_ref[...]], axis=1)

    h = jnp.zeros((B, H), f32)
    c = jnp.zeros((B, H), f32)
    outs = []
    for t in range(T):
        feat_wait(t % 2)
        xw1_t = _D00(feat_buf[t % 2], w1t_ref[...], ((1,), (1,)))
        if t + 2 < T:
            feat_start(t % 2, t + 2)
        h1_t = jnp.maximum(
            _D01(a_ref[t], xw1_t, ((1,), (0,))) + b1_ref[...], 0.0)
        hw_t = _D02(h1_t, w2_ref[...], ((1,), (0,)))
        h2_t = _D03(a_ref[t], hw_t, ((1,), (0,))) + b2_ref[...]
        tiled = _D04(h2_t, ktile, ((1,), (0,))) * mask
        flat_t = _D05(ones_row, tiled, ((1,), (0,)))
        gnn_t = jnp.maximum(_D06(flat_t, fcwt_ref[...], ((1,), (1,)))
                            + fcb_ref[...], 0.0)
        zg_t = _D07(gnn_t, wg, ((1,), (0,))) + b4
        zx_t = _D08(xt_ref[t], wx, ((0,), (0,)))

        z = zx_t + zg_t + _D09(h, wh, ((1,), (0,)))
        s = jax.nn.sigmoid(z[:, :3 * H])
        g_t = jnp.tanh(z[:, 3 * H:])
        c = s[:, H:2 * H] * c + s[:, :H] * g_t
        h = s[:, 2 * H:] * jnp.tanh(c)
        outs.append(h)
    out_ref[...] = jnp.stack(outs, axis=0)
    c_ref[...] = c
    hN_ref[...] = h


def kernel(x_seq, a_seq, feat_seq, w1, b1, w2, b2, fcw_flat, fcb,
           wii, wgi, whi, bi, wif, wgf, whf, bf,
           wig, wgg, whg, bg, wio, wgo, who, bo):
    T, B, I = x_seq.shape
    _, N, F = feat_seq.shape
    GH = w2.shape[0]
    H = whi.shape[0]
    f32 = jnp.float32

    xt = jnp.transpose(x_seq, (0, 2, 1))
    w1t = jnp.transpose(w1)
    fcwt = jnp.transpose(fcw_flat)

    def full(arr):
        return pl.BlockSpec(arr.shape, lambda j, _nd=arr.ndim: (0,) * _nd)

    out, c_T, h_T = pl.pallas_call(
        functools.partial(_fused_kernel, n_nodes=N, gh=GH, seq_len=T,
                          batch=B, hidden=H, in_dim=I),
        out_shape=(
            jax.ShapeDtypeStruct((T, B, H), f32),
            jax.ShapeDtypeStruct((B, H), f32),
            jax.ShapeDtypeStruct((B, H), f32),
        ),
        grid=(1,),
        in_specs=[
            full(xt), full(a_seq),
            pl.BlockSpec(memory_space=pltpu.MemorySpace.HBM),
            full(w1t), full(b1), full(w2), full(b2), full(fcwt), full(fcb),
            full(wgi), full(wgf), full(wgo), full(wgg),
            full(bi), full(bf), full(bo), full(bg),
            full(wii), full(wif), full(wio), full(wig),
            full(whi), full(whf), full(who), full(whg),
        ],
        out_specs=(
            pl.BlockSpec((T, B, H), lambda j: (0, 0, 0)),
            pl.BlockSpec((B, H), lambda j: (0, 0)),
            pl.BlockSpec((B, H), lambda j: (0, 0)),
        ),
        scratch_shapes=[
            pltpu.VMEM((2, N, F), f32),
            pltpu.SemaphoreType.DMA((2,)),
        ],
        compiler_params=pltpu.CompilerParams(
            dimension_semantics=("arbitrary",)),
    )(xt, a_seq, feat_seq, w1t, b1, w2, b2, fcwt, fcb,
      wgi, wgf, wgo, wgg, bi, bf, bo, bg,
      wii, wif, wio, wig, whi, whf, who, whg)

    return out, h_T, c_T


# speedup vs baseline: 1.0414x; 1.0414x over previous
import functools

import jax
import jax.numpy as jnp
from jax.experimental import pallas as pl
from jax.experimental.pallas import tpu as pltpu


def _dg(a, b, dims):
    return (a, b, (dims, ((), ())))


def _D00(a, b, d):
    return jax.lax.dot_general(*_dg(a, b, d), preferred_element_type=jnp.float32)


def _D01(a, b, d):
    return jax.lax.dot_general(*_dg(a, b, d), preferred_element_type=jnp.float32)


def _D02(a, b, d):
    return jax.lax.dot_general(*_dg(a, b, d), preferred_element_type=jnp.float32)


def _D03(a, b, d):
    return jax.lax.dot_general(*_dg(a, b, d), preferred_element_type=jnp.float32)


def _D04(a, b, d):
    return jax.lax.dot_general(*_dg(a, b, d), preferred_element_type=jnp.float32)


def _D05(a, b, d):
    return jax.lax.dot_general(*_dg(a, b, d), preferred_element_type=jnp.float32)


def _D06(a, b, d):
    return jax.lax.dot_general(*_dg(a, b, d), preferred_element_type=jnp.float32)


def _D07(a, b, d):
    return jax.lax.dot_general(*_dg(a, b, d), preferred_element_type=jnp.float32)


def _D08(a, b, d):
    return jax.lax.dot_general(*_dg(a, b, d), preferred_element_type=jnp.float32)


def _D09(a, b, d):
    return jax.lax.dot_general(*_dg(a, b, d), preferred_element_type=jnp.float32)


def _fused_kernel(xt_ref, a_ref, feat_ref,
                  w1t_ref, b1_ref, w2_ref, b2_ref, fcwt_ref, fcb_ref,
                  wgi_ref, wgf_ref, wgo_ref, wgg_ref,
                  bi_ref, bf_ref, bo_ref, bg_ref,
                  wii_ref, wif_ref, wio_ref, wig_ref,
                  whi_ref, whf_ref, who_ref, whg_ref,
                  out_ref, c_ref, hN_ref,
                  feat_buf, feat_sem,
                  *, n_nodes, gh, seq_len, batch, hidden, in_dim):
    f32 = jnp.float32
    N, GH, T, B, H = n_nodes, gh, seq_len, batch, hidden
    NG = N * GH

    def feat_start(slot, t):
        pltpu.make_async_copy(feat_ref.at[t], feat_buf.at[slot],
                              feat_sem.at[slot]).start()

    def feat_wait(slot):
        pltpu.make_async_copy(feat_buf.at[slot], feat_buf.at[slot],
                              feat_sem.at[slot]).wait()

    feat_start(0, 0)
    feat_start(1, 1)

    col = jax.lax.broadcasted_iota(jnp.int32, (N, NG), 1)
    row = jax.lax.broadcasted_iota(jnp.int32, (N, NG), 0)
    mask = (col // GH == row).astype(f32)
    kcol = jax.lax.broadcasted_iota(jnp.int32, (GH, NG), 1)
    krow = jax.lax.broadcasted_iota(jnp.int32, (GH, NG), 0)
    ktile = (kcol % GH == krow).astype(f32)
    ones_row = jnp.ones((1, N), f32)

    wg = jnp.concatenate([wgi_ref[...], wgf_ref[...],
                          wgo_ref[...], wgg_ref[...]], axis=1)
    b4 = jnp.concatenate([bi_ref[...], bf_ref[...],
                          bo_ref[...], bg_ref[...]], axis=1)
    wx = jnp.concatenate([wii_ref[...], wif_ref[...],
                          wio_ref[...], wig_ref[...]], axis=1)
    wh = jnp.concatenate([whi_ref[...], whf_ref[...],
                          who_ref[...], whg_ref[...]], axis=1)

    zgs, zxs = [], []
    for t in range(T):
        feat_wait(t % 2)
        xw1_t = _D00(feat_buf[t % 2], w1t_ref[...], ((1,), (1,)))
        if t + 2 < T:
            feat_start(t % 2, t + 2)
        h1_t = jnp.maximum(
            _D01(a_ref[t], xw1_t, ((1,), (0,))) + b1_ref[...], 0.0)
        hw_t = _D02(h1_t, w2_ref[...], ((1,), (0,)))
        h2_t = _D03(a_ref[t], hw_t, ((1,), (0,))) + b2_ref[...]
        tiled = _D04(h2_t, ktile, ((1,), (0,))) * mask
        flat_t = _D05(ones_row, tiled, ((1,), (0,)))
        gnn_t = jnp.maximum(_D06(flat_t, fcwt_ref[...], ((1,), (1,)))
                            + fcb_ref[...], 0.0)
        zgs.append(_D07(gnn_t, wg, ((1,), (0,))) + b4)
        zxs.append(_D08(xt_ref[t], wx, ((0,), (0,))))

    h = jnp.zeros((B, H), f32)
    c = jnp.zeros((B, H), f32)
    outs = []
    for t in range(T):
        z = zxs[t] + zgs[t] + _D09(h, wh, ((1,), (0,)))
        s = jax.nn.sigmoid(z[:, :3 * H])
        g_t = jnp.tanh(z[:, 3 * H:])
        c = s[:, H:2 * H] * c + s[:, :H] * g_t
        h = s[:, 2 * H:] * jnp.tanh(c)
        outs.append(h)
    out_ref[...] = jnp.stack(outs, axis=0)
    c_ref[...] = c
    hN_ref[...] = h


def kernel(x_seq, a_seq, feat_seq, w1, b1, w2, b2, fcw_flat, fcb,
           wii, wgi, whi, bi, wif, wgf, whf, bf,
           wig, wgg, whg, bg, wio, wgo, who, bo):
    T, B, I = x_seq.shape
    _, N, F = feat_seq.shape
    GH = w2.shape[0]
    H = whi.shape[0]
    f32 = jnp.float32

    xt = jnp.transpose(x_seq, (0, 2, 1))
    w1t = jnp.transpose(w1)
    fcwt = jnp.transpose(fcw_flat)

    def full(arr):
        return pl.BlockSpec(arr.shape, lambda j, _nd=arr.ndim: (0,) * _nd)

    out, c_T, h_T = pl.pallas_call(
        functools.partial(_fused_kernel, n_nodes=N, gh=GH, seq_len=T,
                          batch=B, hidden=H, in_dim=I),
        out_shape=(
            jax.ShapeDtypeStruct((T, B, H), f32),
            jax.ShapeDtypeStruct((B, H), f32),
            jax.ShapeDtypeStruct((B, H), f32),
        ),
        grid=(1,),
        in_specs=[
            full(xt), full(a_seq),
            pl.BlockSpec(memory_space=pltpu.MemorySpace.HBM),
            full(w1t), full(b1), full(w2), full(b2), full(fcwt), full(fcb),
            full(wgi), full(wgf), full(wgo), full(wgg),
            full(bi), full(bf), full(bo), full(bg),
            full(wii), full(wif), full(wio), full(wig),
            full(whi), full(whf), full(who), full(whg),
        ],
        out_specs=(
            pl.BlockSpec((T, B, H), lambda j: (0, 0, 0)),
            pl.BlockSpec((B, H), lambda j: (0, 0)),
            pl.BlockSpec((B, H), lambda j: (0, 0)),
        ),
        scratch_shapes=[
            pltpu.VMEM((2, N, F), f32),
            pltpu.SemaphoreType.DMA((2,)),
        ],
        compiler_params=pltpu.CompilerParams(
            dimension_semantics=("arbitrary",)),
    )(xt, a_seq, feat_seq, w1t, b1, w2, b2, fcwt, fcb,
      wgi, wgf, wgo, wgg, bi, bf, bo, bg,
      wii, wif, wio, wig, whi, whf, who, whg)

    return out, h_T, c_T


# speedup vs baseline: 1.3681x; 1.3137x over previous
import functools

import jax
import jax.numpy as jnp
from jax.experimental import pallas as pl
from jax.experimental.pallas import tpu as pltpu


def _dg(a, b, dims):
    return (a, b, (dims, ((), ())))


def _D00(a, b, d):
    return jax.lax.dot_general(*_dg(a, b, d), preferred_element_type=jnp.float32)


def _D01(a, b, d):
    return jax.lax.dot_general(*_dg(a, b, d), preferred_element_type=jnp.float32)


def _D02(a, b, d):
    return jax.lax.dot_general(*_dg(a, b, d), preferred_element_type=jnp.float32)


def _D03(a, b, d):
    return jax.lax.dot_general(*_dg(a, b, d), preferred_element_type=jnp.float32)


def _D04(a, b, d):
    return jax.lax.dot_general(*_dg(a, b, d), preferred_element_type=jnp.float32)


def _D05(a, b, d):
    return jax.lax.dot_general(*_dg(a, b, d), preferred_element_type=jnp.float32)


def _D06(a, b, d):
    return jax.lax.dot_general(*_dg(a, b, d), preferred_element_type=jnp.float32)


def _D07(a, b, d):
    return jax.lax.dot_general(*_dg(a, b, d), preferred_element_type=jnp.float32)


def _D08(a, b, d):
    return jax.lax.dot_general(*_dg(a, b, d), preferred_element_type=jnp.float32)


def _D09(a, b, d):
    return jax.lax.dot_general(*_dg(a, b, d), preferred_element_type=jnp.float32)


def _fused_kernel(xt_ref, a_ref, feat_ref,
                  w1t_ref, b1_ref, w2_ref, b2_ref, fcwt_ref, fcb_ref,
                  wgi_ref, wgf_ref, wgo_ref, wgg_ref,
                  bi_ref, bf_ref, bo_ref, bg_ref,
                  wii_ref, wif_ref, wio_ref, wig_ref,
                  whi_ref, whf_ref, who_ref, whg_ref,
                  out_ref, c_ref, hN_ref,
                  feat_buf, feat_sem, fcw_buf, x_buf, aux_sem,
                  *, n_nodes, gh, seq_len, batch, hidden, in_dim):
    f32 = jnp.float32
    N, GH, T, B, H = n_nodes, gh, seq_len, batch, hidden
    NG = N * GH

    def feat_start(slot, t):
        pltpu.make_async_copy(feat_ref.at[t], feat_buf.at[slot],
                              feat_sem.at[slot]).start()

    def feat_wait(slot):
        pltpu.make_async_copy(feat_buf.at[slot], feat_buf.at[slot],
                              feat_sem.at[slot]).wait()

    for k in range(min(4, T)):
        feat_start(k, k)
    fcw_cp = pltpu.make_async_copy(fcwt_ref, fcw_buf, aux_sem.at[0])
    fcw_cp.start()
    x_cp = pltpu.make_async_copy(xt_ref, x_buf, aux_sem.at[1])
    x_cp.start()

    col = jax.lax.broadcasted_iota(jnp.int32, (N, NG), 1)
    row = jax.lax.broadcasted_iota(jnp.int32, (N, NG), 0)
    mask = (col // GH == row).astype(f32)
    kcol = jax.lax.broadcasted_iota(jnp.int32, (GH, NG), 1)
    krow = jax.lax.broadcasted_iota(jnp.int32, (GH, NG), 0)
    ktile = (kcol % GH == krow).astype(f32)
    ones_row = jnp.ones((1, N), f32)

    h1s = []
    for t in range(T):
        feat_wait(t % 4)
        xw1_t = _D00(feat_buf[t % 4], w1t_ref[...], ((1,), (1,)))
        if t + 4 < T:
            feat_start(t % 4, t + 4)
        h1s.append(jnp.maximum(
            _D01(a_ref[t], xw1_t, ((1,), (0,))) + b1_ref[...], 0.0))
    h1 = jnp.concatenate(h1s, axis=0)
    hw = _D02(h1, w2_ref[...], ((1,), (0,)))

    flats = []
    for t in range(T):
        h2_t = (_D03(a_ref[t], hw[t * N:(t + 1) * N, :], ((1,), (0,)))
                + b2_ref[...])
        tiled = _D04(h2_t, ktile, ((1,), (0,))) * mask
        flats.append(_D05(ones_row, tiled, ((1,), (0,))))
    flat = jnp.concatenate(flats, axis=0)

    fcw_cp.wait()
    gnn = jnp.maximum(_D06(flat, fcw_buf[...], ((1,), (1,)))
                      + fcb_ref[...], 0.0)
    wg = jnp.concatenate([wgi_ref[...], wgf_ref[...],
                          wgo_ref[...], wgg_ref[...]], axis=1)
    b4 = jnp.concatenate([bi_ref[...], bf_ref[...],
                          bo_ref[...], bg_ref[...]], axis=1)
    zg = _D07(gnn, wg, ((1,), (0,))) + b4

    wx = jnp.concatenate([wii_ref[...], wif_ref[...],
                          wio_ref[...], wig_ref[...]], axis=1)
    wh = jnp.concatenate([whi_ref[...], whf_ref[...],
                          who_ref[...], whg_ref[...]], axis=1)
    x_cp.wait()
    zxs = [_D08(x_buf[t], wx, ((0,), (0,))) for t in range(T)]

    h = jnp.zeros((B, H), f32)
    c = jnp.zeros((B, H), f32)
    outs = []
    for t in range(T):
        z = zxs[t] + zg[t:t + 1, :] + _D09(h, wh, ((1,), (0,)))
        s = jax.nn.sigmoid(z[:, :3 * H])
        g_t = jnp.tanh(z[:, 3 * H:])
        c = s[:, H:2 * H] * c + s[:, :H] * g_t
        h = s[:, 2 * H:] * jnp.tanh(c)
        outs.append(h)
    out_ref[...] = jnp.stack(outs, axis=0)
    c_ref[...] = c
    hN_ref[...] = h


def kernel(x_seq, a_seq, feat_seq, w1, b1, w2, b2, fcw_flat, fcb,
           wii, wgi, whi, bi, wif, wgf, whf, bf,
           wig, wgg, whg, bg, wio, wgo, who, bo):
    T, B, I = x_seq.shape
    _, N, F = feat_seq.shape
    GH = w2.shape[0]
    H = whi.shape[0]
    f32 = jnp.float32

    xt = jnp.transpose(x_seq, (0, 2, 1))
    w1t = jnp.transpose(w1)
    fcwt = jnp.transpose(fcw_flat)

    def full(arr):
        return pl.BlockSpec(arr.shape, lambda j, _nd=arr.ndim: (0,) * _nd)

    out, c_T, h_T = pl.pallas_call(
        functools.partial(_fused_kernel, n_nodes=N, gh=GH, seq_len=T,
                          batch=B, hidden=H, in_dim=I),
        out_shape=(
            jax.ShapeDtypeStruct((T, B, H), f32),
            jax.ShapeDtypeStruct((B, H), f32),
            jax.ShapeDtypeStruct((B, H), f32),
        ),
        grid=(1,),
        in_specs=[
            pl.BlockSpec(memory_space=pltpu.MemorySpace.HBM),
            full(a_seq),
            pl.BlockSpec(memory_space=pltpu.MemorySpace.HBM),
            full(w1t), full(b1), full(w2), full(b2),
            pl.BlockSpec(memory_space=pltpu.MemorySpace.HBM),
            full(fcb),
            full(wgi), full(wgf), full(wgo), full(wgg),
            full(bi), full(bf), full(bo), full(bg),
            full(wii), full(wif), full(wio), full(wig),
            full(whi), full(whf), full(who), full(whg),
        ],
        out_specs=(
            pl.BlockSpec((T, B, H), lambda j: (0, 0, 0)),
            pl.BlockSpec((B, H), lambda j: (0, 0)),
            pl.BlockSpec((B, H), lambda j: (0, 0)),
        ),
        scratch_shapes=[
            pltpu.VMEM((4, N, F), f32),
            pltpu.SemaphoreType.DMA((4,)),
            pltpu.VMEM(fcwt.shape, f32),
            pltpu.VMEM(xt.shape, f32),
            pltpu.SemaphoreType.DMA((2,)),
        ],
        compiler_params=pltpu.CompilerParams(
            dimension_semantics=("arbitrary",)),
    )(xt, a_seq, feat_seq, w1t, b1, w2, b2, fcwt, fcb,
      wgi, wgf, wgo, wgg, bi, bf, bo, bg,
      wii, wif, wio, wig, whi, whf, who, whg)

    return out, h_T, c_T
